# baseline (speedup 1.0000x reference)
import functools

import jax
import jax.numpy as jnp
from jax import lax
from jax.experimental import pallas as pl
from jax.experimental.pallas import tpu as pltpu
from jax.experimental.pallas import tpu_sc as plsc

N = 10000
E = 160000
H = 128
H2 = 256
L = 4
NL = 5
OUT = 128

NP = 10240
DUMMY = N
NTILE = 16
NB = 80
CB = 40
B = 128
EPT = NB * B
E_PAD = NTILE * EPT
RPT = NP // NTILE
BN_ROWS = 2000
GRID = N // BN_ROWS

def _sc_segsum_body(h_hbm, src_hbm, dst_hbm, out_hbm, src_v, dst_v, rows_v0,
                    rows_v1, agg_sh, sg0, sg1, ss0, ss1):
    c = lax.axis_index("c")
    s = lax.axis_index("s")

    pltpu.sync_copy(src_hbm.at[s], src_v)
    pltpu.sync_copy(dst_hbm.at[s], dst_v)

    @pl.loop(0, B)
    def _(r):
        @pl.loop(0, H, step=16)
        def _(k):
            rows_v0.at[r][pl.ds(k, 16)] = jnp.zeros((16,), jnp.float32)

    @pl.loop(0, RPT, step=B)
    def _(r0):
        pltpu.sync_copy(rows_v0, agg_sh.at[pl.ds(s * RPT + r0, B)])

    plsc.subcore_barrier()

    plane = h_hbm.at[c]

    def g(j, buf, sem):
        return pltpu.make_async_copy(plane.at[src_v.at[j]], buf, sem)

    def sc(j, buf, sem):
        return pltpu.make_async_copy(buf, agg_sh.at[dst_v.at[j]], sem)

    for ch in range(NB // CB):
        pltpu.sync_copy(src_hbm.at[s * (NB // CB) + ch], src_v)
        pltpu.sync_copy(dst_hbm.at[s * (NB // CB) + ch], dst_v)

        g(0, rows_v0, sg0).start()

        @pl.loop(0, CB, step=2)
        def _(j):
            g(j, rows_v0, sg0).wait()

            @pl.when(j > 0)
            def _():
                sc(j - 1, rows_v1, ss1).wait()

            g(j + 1, rows_v1, sg1).start()
            sc(j, rows_v0, ss0).start(add=True)
            g(j + 1, rows_v1, sg1).wait()

            @pl.when(j + 2 < CB)
            def _():
                sc(j, rows_v0, ss0).wait()
                g(j + 2, rows_v0, sg0).start()

            sc(j + 1, rows_v1, ss1).start(add=True)

        sc(CB - 2, rows_v0, ss0).wait()
        sc(CB - 1, rows_v1, ss1).wait()

    plsc.subcore_barrier()

    pltpu.sync_copy(agg_sh.at[pl.ds(s * RPT, RPT)],
                    out_hbm.at[c].at[pl.ds(s * RPT, RPT)])


@functools.cache
def _sc_segsum_kernel():
    mesh = plsc.VectorSubcoreMesh(core_axis_name="c", subcore_axis_name="s",
                                  num_cores=2, num_subcores=NTILE)
    return pl.kernel(
        _sc_segsum_body,
        out_type=jax.ShapeDtypeStruct((2, NP, H), jnp.float32),
        mesh=mesh,
        scratch_types=[
            pltpu.VMEM((CB, B), jnp.int32),
            pltpu.VMEM((CB, B), jnp.int32),
            pltpu.VMEM((B, H), jnp.float32),
            pltpu.VMEM((B, H), jnp.float32),
            pltpu.VMEM_SHARED((NP, H), jnp.float32),
            pltpu.SemaphoreType.DMA,
            pltpu.SemaphoreType.DMA,
            pltpu.SemaphoreType.DMA,
            pltpu.SemaphoreType.DMA,
        ],
    )


def _sc_segsum(h, src_p, dst_p):
    return _sc_segsum_kernel()(h, src_p, dst_p)


BE = 64
NBE = RPT // BE


def _sc_embed_body(state_hbm, emb_hbm, c_hbm, out_hbm, idx_v, rows_v):
    c = lax.axis_index("c")
    s = lax.axis_index("s")
    row0 = s * RPT

    @pl.when(c == 0)
    def _():
        pltpu.sync_copy(state_hbm.at[s], idx_v)

        @pl.loop(0, NBE)
        def _(b):
            pltpu.sync_copy(emb_hbm.at[idx_v.at[b]], rows_v)
            pltpu.sync_copy(rows_v,
                            out_hbm.at[0].at[pl.ds(row0 + b * BE, BE)])

    @pl.when(c == 1)
    def _():
        pltpu.sync_copy(c_hbm, rows_v.at[pl.ds(0, 1)])

        @pl.loop(1, BE)
        def _(r):
            @pl.loop(0, H, step=16)
            def _(k):
                rows_v.at[r][pl.ds(k, 16)] = rows_v.at[0][pl.ds(k, 16)]

        @pl.loop(0, NBE)
        def _(b):
            pltpu.sync_copy(rows_v,
                            out_hbm.at[1].at[pl.ds(row0 + b * BE, BE)])


@functools.cache
def _sc_embed_kernel():
    mesh = plsc.VectorSubcoreMesh(core_axis_name="c", subcore_axis_name="s",
                                  num_cores=2, num_subcores=NTILE)
    return pl.kernel(
        _sc_embed_body,
        out_type=jax.ShapeDtypeStruct((2, NP, H), jnp.float32),
        mesh=mesh,
        scratch_types=[
            pltpu.VMEM((NBE, BE), jnp.int32),
            pltpu.VMEM((BE, H), jnp.float32),
        ],
    )


def _sc_embed(state_rs, emb, c2d):
    return _sc_embed_kernel()(state_rs, emb, c2d)


_P = jax.lax.Precision.DEFAULT


def _dot(a, b):
    return jnp.dot(a, b, precision=_P, preferred_element_type=jnp.float32)


def _accum_stats(i, t, st_ref):
    @pl.when(i == 0)
    def _():
        mu0 = jnp.mean(t, axis=0)
        ctr = t - mu0[None, :]
        st_ref[...] = jnp.concatenate(
            [jnp.sum(ctr, axis=0)[None, :],
             jnp.sum(ctr * ctr, axis=0)[None, :],
             mu0[None, :]], axis=0)

    @pl.when(i > 0)
    def _():
        mu0 = st_ref[2]
        ctr = t - mu0[None, :]
        st_ref[...] += jnp.concatenate(
            [jnp.sum(ctr, axis=0)[None, :],
             jnp.sum(ctr * ctr, axis=0)[None, :],
             jnp.zeros((1, H2), jnp.float32)], axis=0)


def _tc_a_body(h_ref, agg_ref, wa_ref, t1_ref, st_ref):
    i = pl.program_id(0)
    rst = jnp.concatenate(
        [h_ref[0] + agg_ref[0], h_ref[1] + agg_ref[1]], axis=1)
    t1 = _dot(rst, wa_ref[...])
    t1_ref[...] = t1
    _accum_stats(i, t1, st_ref)


def _tc_a(h, agg, wa):
    return pl.pallas_call(
        _tc_a_body,
        grid=(GRID,),
        in_specs=[
            pl.BlockSpec((2, BN_ROWS, H), lambda i: (0, i, 0)),
            pl.BlockSpec((2, BN_ROWS, H), lambda i: (0, i, 0)),
            pl.BlockSpec((H2, H2), lambda i: (0, 0)),
        ],
        out_specs=[
            pl.BlockSpec((BN_ROWS, H2), lambda i: (i, 0)),
            pl.BlockSpec((3, H2), lambda i: (0, 0)),
        ],
        out_shape=[
            jax.ShapeDtypeStruct((N, H2), jnp.float32),
            jax.ShapeDtypeStruct((3, H2), jnp.float32),
        ],
    )(h, agg, wa)


def _bn_relu(t, st, gamma, beta):
    d1 = st[0] * (1.0 / N)
    m = st[2] + d1
    v = st[1] * (1.0 / N) - d1 * d1
    inv = lax.rsqrt(v + 1e-5)
    return jnp.maximum((t - m[None, :]) * (gamma * inv)[None, :] + beta[None, :],
                       0.0)


def _tc_b_body(t1_ref, st_ref, ga_ref, ba_ref, wb_ref, t2_ref, st2_ref):
    i = pl.program_id(0)
    z = _bn_relu(t1_ref[...], st_ref[...], ga_ref[0], ba_ref[0])
    t2 = _dot(z, wb_ref[...])
    t2_ref[...] = t2
    _accum_stats(i, t2, st2_ref)


def _tc_b(t1, st1, ga, ba, wb):
    return pl.pallas_call(
        _tc_b_body,
        grid=(GRID,),
        in_specs=[
            pl.BlockSpec((BN_ROWS, H2), lambda i: (i, 0)),
            pl.BlockSpec((3, H2), lambda i: (0, 0)),
            pl.BlockSpec((1, H2), lambda i: (0, 0)),
            pl.BlockSpec((1, H2), lambda i: (0, 0)),
            pl.BlockSpec((H2, H2), lambda i: (0, 0)),
        ],
        out_specs=[
            pl.BlockSpec((BN_ROWS, H2), lambda i: (i, 0)),
            pl.BlockSpec((3, H2), lambda i: (0, 0)),
        ],
        out_shape=[
            jax.ShapeDtypeStruct((N, H2), jnp.float32),
            jax.ShapeDtypeStruct((3, H2), jnp.float32),
        ],
    )(t1, st1, ga, ba, wb)


def _tc_c_body(t2_ref, st_ref, go_ref, bo_ref, h_ref):
    h = _bn_relu(t2_ref[...], st_ref[...], go_ref[0], bo_ref[0])
    h_ref[0] = h[:, :H]
    h_ref[1] = h[:, H:]


def _tc_c(t2, st2, go, bo):
    return pl.pallas_call(
        _tc_c_body,
        grid=(GRID,),
        in_specs=[
            pl.BlockSpec((BN_ROWS, H2), lambda i: (i, 0)),
            pl.BlockSpec((3, H2), lambda i: (0, 0)),
            pl.BlockSpec((1, H2), lambda i: (0, 0)),
            pl.BlockSpec((1, H2), lambda i: (0, 0)),
        ],
        out_specs=pl.BlockSpec((2, BN_ROWS, H), lambda i: (0, i, 0)),
        out_shape=jax.ShapeDtypeStruct((2, NP, H), jnp.float32),
    )(t2, st2, go, bo)


def _tc_readout_body(h0, h1, h2, h3, h4, wr1_ref, br1_ref, wr2_ref, br2_ref,
                     o_ref):
    acc = jnp.broadcast_to(br1_ref[0], (BN_ROWS, H2))
    for k, h_ref in enumerate((h0, h1, h2, h3, h4)):
        hcat = jnp.concatenate([h_ref[0], h_ref[1]], axis=1)
        acc = acc + _dot(hcat, wr1_ref[k])
    o_ref[...] = _dot(jnp.maximum(acc, 0.0), wr2_ref[...]) + br2_ref[0]


def _tc_readout(hs, wr1r, br1, wr2, br2):
    hspec = pl.BlockSpec((2, BN_ROWS, H), lambda i: (0, i, 0))
    return pl.pallas_call(
        _tc_readout_body,
        grid=(GRID,),
        in_specs=[hspec] * NL + [
            pl.BlockSpec((NL, H2, H2), lambda i: (0, 0, 0)),
            pl.BlockSpec((1, H2), lambda i: (0, 0)),
            pl.BlockSpec((H2, OUT), lambda i: (0, 0)),
            pl.BlockSpec((1, OUT), lambda i: (0, 0)),
        ],
        out_specs=pl.BlockSpec((BN_ROWS, OUT), lambda i: (i, 0)),
        out_shape=jax.ShapeDtypeStruct((N, OUT), jnp.float32),
    )(*hs, wr1r, br1, wr2, br2)


def kernel(state, edge_index, c, emb, Wa, ga, ba, Wb, go, bo, Wr1, br1, Wr2,
           br2):
    src = edge_index[0]
    dst = edge_index[1]
    nch = NB // CB
    src_p = jnp.concatenate(
        [src, jnp.zeros((E_PAD - E,), jnp.int32)]).reshape(NTILE * nch, CB, B)
    dst_p = jnp.concatenate(
        [dst, jnp.full((E_PAD - E,), DUMMY, jnp.int32)]).reshape(
            NTILE * nch, CB, B)

    state_rs = jnp.concatenate(
        [state, jnp.zeros((NP - N,), jnp.int32)]).reshape(NTILE, NBE, BE)
    c2d = c.reshape(1, H)

    h = _sc_embed(state_rs, emb, c2d)

    hs = [h]
    for i in range(L):
        agg = _sc_segsum(h, src_p, dst_p)
        t1, st1 = _tc_a(h, agg, Wa[i])
        t2, st2 = _tc_b(t1, st1, ga[i].reshape(1, H2), ba[i].reshape(1, H2),
                        Wb[i])
        h = _tc_c(t2, st2, go[i].reshape(1, H2), bo[i].reshape(1, H2))
        hs.append(h)

    wr1r = Wr1.reshape(NL, H2, H2)
    return _tc_readout(hs, wr1r, br1.reshape(1, H2), Wr2, br2.reshape(1, OUT))


# speedup vs baseline: 1.1802x; 1.1802x over previous
import functools

import jax
import jax.numpy as jnp
from jax import lax
from jax.experimental import pallas as pl
from jax.experimental.pallas import tpu as pltpu
from jax.experimental.pallas import tpu_sc as plsc

N = 10000
E = 160000
H = 128
H2 = 256
L = 4
NL = 5
OUT = 128

NP = 10240
DUMMY = N
NTILE = 16
NB = 79
B = 128
EPT = NB * B
E_PAD = NTILE * EPT
RPT = NP // NTILE
BN_ROWS = 2000
GRID = N // BN_ROWS

def _sc_segsum_body(h_hbm, src_hbm, dst_hbm, out_hbm, src_v, dst_v, rows_v,
                    agg_sh):
    c = lax.axis_index("c")
    s = lax.axis_index("s")

    pltpu.sync_copy(src_hbm.at[s], src_v)
    pltpu.sync_copy(dst_hbm.at[s], dst_v)

    @pl.loop(0, B)
    def _(r):
        @pl.loop(0, H, step=16)
        def _(k):
            rows_v.at[r][pl.ds(k, 16)] = jnp.zeros((16,), jnp.float32)

    @pl.loop(0, RPT, step=B)
    def _(r0):
        pltpu.sync_copy(rows_v, agg_sh.at[pl.ds(s * RPT + r0, B)])

    plsc.subcore_barrier()

    plane = h_hbm.at[c]

    @pl.loop(0, NB)
    def _(j):
        pltpu.sync_copy(plane.at[src_v.at[j]], rows_v)
        pltpu.sync_copy(rows_v, agg_sh.at[dst_v.at[j]], add=True)

    plsc.subcore_barrier()

    pltpu.sync_copy(agg_sh.at[pl.ds(s * RPT, RPT)],
                    out_hbm.at[c].at[pl.ds(s * RPT, RPT)])


@functools.cache
def _sc_segsum_kernel():
    mesh = plsc.VectorSubcoreMesh(core_axis_name="c", subcore_axis_name="s",
                                  num_cores=2, num_subcores=NTILE)
    return pl.kernel(
        _sc_segsum_body,
        out_type=jax.ShapeDtypeStruct((2, NP, H), jnp.float32),
        mesh=mesh,
        scratch_types=[
            pltpu.VMEM((NB, B), jnp.int32),
            pltpu.VMEM((NB, B), jnp.int32),
            pltpu.VMEM((B, H), jnp.float32),
            pltpu.VMEM_SHARED((NP, H), jnp.float32),
        ],
    )


def _sc_segsum(h, src_p, dst_p):
    return _sc_segsum_kernel()(h, src_p, dst_p)


BE = 64
NBE = RPT // BE


def _sc_embed_body(state_hbm, emb_hbm, c_hbm, out_hbm, idx_v, rows_v):
    c = lax.axis_index("c")
    s = lax.axis_index("s")
    row0 = s * RPT

    @pl.when(c == 0)
    def _():
        pltpu.sync_copy(state_hbm.at[s], idx_v)

        @pl.loop(0, NBE)
        def _(b):
            pltpu.sync_copy(emb_hbm.at[idx_v.at[b]], rows_v)
            pltpu.sync_copy(rows_v,
                            out_hbm.at[0].at[pl.ds(row0 + b * BE, BE)])

    @pl.when(c == 1)
    def _():
        pltpu.sync_copy(c_hbm, rows_v.at[pl.ds(0, 1)])

        @pl.loop(1, BE)
        def _(r):
            @pl.loop(0, H, step=16)
            def _(k):
                rows_v.at[r][pl.ds(k, 16)] = rows_v.at[0][pl.ds(k, 16)]

        @pl.loop(0, NBE)
        def _(b):
            pltpu.sync_copy(rows_v,
                            out_hbm.at[1].at[pl.ds(row0 + b * BE, BE)])


@functools.cache
def _sc_embed_kernel():
    mesh = plsc.VectorSubcoreMesh(core_axis_name="c", subcore_axis_name="s",
                                  num_cores=2, num_subcores=NTILE)
    return pl.kernel(
        _sc_embed_body,
        out_type=jax.ShapeDtypeStruct((2, NP, H), jnp.float32),
        mesh=mesh,
        scratch_types=[
            pltpu.VMEM((NBE, BE), jnp.int32),
            pltpu.VMEM((BE, H), jnp.float32),
        ],
    )


def _sc_embed(state_rs, emb, c2d):
    return _sc_embed_kernel()(state_rs, emb, c2d)


_P = jax.lax.Precision.DEFAULT


def _dot(a, b):
    return jnp.dot(a, b, precision=_P, preferred_element_type=jnp.float32)


def _accum_stats(i, t, st_ref):
    @pl.when(i == 0)
    def _():
        mu0 = jnp.mean(t, axis=0)
        ctr = t - mu0[None, :]
        st_ref[...] = jnp.concatenate(
            [jnp.sum(ctr, axis=0)[None, :],
             jnp.sum(ctr * ctr, axis=0)[None, :],
             mu0[None, :]], axis=0)

    @pl.when(i > 0)
    def _():
        mu0 = st_ref[2]
        ctr = t - mu0[None, :]
        st_ref[...] += jnp.concatenate(
            [jnp.sum(ctr, axis=0)[None, :],
             jnp.sum(ctr * ctr, axis=0)[None, :],
             jnp.zeros((1, H2), jnp.float32)], axis=0)


def _tc_a_body(h_ref, agg_ref, wa_ref, t1_ref, st_ref):
    i = pl.program_id(0)
    rst = jnp.concatenate(
        [h_ref[0] + agg_ref[0], h_ref[1] + agg_ref[1]], axis=1)
    t1 = _dot(rst, wa_ref[...])
    t1_ref[...] = t1
    _accum_stats(i, t1, st_ref)


def _tc_a(h, agg, wa):
    return pl.pallas_call(
        _tc_a_body,
        grid=(GRID,),
        in_specs=[
            pl.BlockSpec((2, BN_ROWS, H), lambda i: (0, i, 0)),
            pl.BlockSpec((2, BN_ROWS, H), lambda i: (0, i, 0)),
            pl.BlockSpec((H2, H2), lambda i: (0, 0)),
        ],
        out_specs=[
            pl.BlockSpec((BN_ROWS, H2), lambda i: (i, 0)),
            pl.BlockSpec((3, H2), lambda i: (0, 0)),
        ],
        out_shape=[
            jax.ShapeDtypeStruct((N, H2), jnp.float32),
            jax.ShapeDtypeStruct((3, H2), jnp.float32),
        ],
    )(h, agg, wa)


def _bn_relu(t, st, gamma, beta):
    d1 = st[0] * (1.0 / N)
    m = st[2] + d1
    v = st[1] * (1.0 / N) - d1 * d1
    inv = lax.rsqrt(v + 1e-5)
    return jnp.maximum((t - m[None, :]) * (gamma * inv)[None, :] + beta[None, :],
                       0.0)


def _tc_b_body(t1_ref, st_ref, ga_ref, ba_ref, wb_ref, t2_ref, st2_ref):
    i = pl.program_id(0)
    z = _bn_relu(t1_ref[...], st_ref[...], ga_ref[0], ba_ref[0])
    t2 = _dot(z, wb_ref[...])
    t2_ref[...] = t2
    _accum_stats(i, t2, st2_ref)


def _tc_b(t1, st1, ga, ba, wb):
    return pl.pallas_call(
        _tc_b_body,
        grid=(GRID,),
        in_specs=[
            pl.BlockSpec((BN_ROWS, H2), lambda i: (i, 0)),
            pl.BlockSpec((3, H2), lambda i: (0, 0)),
            pl.BlockSpec((1, H2), lambda i: (0, 0)),
            pl.BlockSpec((1, H2), lambda i: (0, 0)),
            pl.BlockSpec((H2, H2), lambda i: (0, 0)),
        ],
        out_specs=[
            pl.BlockSpec((BN_ROWS, H2), lambda i: (i, 0)),
            pl.BlockSpec((3, H2), lambda i: (0, 0)),
        ],
        out_shape=[
            jax.ShapeDtypeStruct((N, H2), jnp.float32),
            jax.ShapeDtypeStruct((3, H2), jnp.float32),
        ],
    )(t1, st1, ga, ba, wb)


def _tc_c_body(t2_ref, st_ref, go_ref, bo_ref, h_ref):
    h = _bn_relu(t2_ref[...], st_ref[...], go_ref[0], bo_ref[0])
    h_ref[0] = h[:, :H]
    h_ref[1] = h[:, H:]


def _tc_c(t2, st2, go, bo):
    return pl.pallas_call(
        _tc_c_body,
        grid=(GRID,),
        in_specs=[
            pl.BlockSpec((BN_ROWS, H2), lambda i: (i, 0)),
            pl.BlockSpec((3, H2), lambda i: (0, 0)),
            pl.BlockSpec((1, H2), lambda i: (0, 0)),
            pl.BlockSpec((1, H2), lambda i: (0, 0)),
        ],
        out_specs=pl.BlockSpec((2, BN_ROWS, H), lambda i: (0, i, 0)),
        out_shape=jax.ShapeDtypeStruct((2, NP, H), jnp.float32),
    )(t2, st2, go, bo)


def _tc_p0_body(h_ref, wr1_ref, br1_ref, p_ref):
    hcat = jnp.concatenate([h_ref[0], h_ref[1]], axis=1)
    p_ref[...] = _dot(hcat, wr1_ref[...]) + br1_ref[0]


def _tc_p0(h, wr1, br1):
    return pl.pallas_call(
        _tc_p0_body,
        grid=(GRID,),
        in_specs=[
            pl.BlockSpec((2, BN_ROWS, H), lambda i: (0, i, 0)),
            pl.BlockSpec((H2, H2), lambda i: (0, 0)),
            pl.BlockSpec((1, H2), lambda i: (0, 0)),
        ],
        out_specs=pl.BlockSpec((BN_ROWS, H2), lambda i: (i, 0)),
        out_shape=jax.ShapeDtypeStruct((N, H2), jnp.float32),
    )(h, wr1, br1)


def _tc_pacc_body(p_in_ref, h_ref, wr1_ref, p_ref):
    hcat = jnp.concatenate([h_ref[0], h_ref[1]], axis=1)
    p_ref[...] = p_in_ref[...] + _dot(hcat, wr1_ref[...])


def _tc_pacc(p_in, h, wr1):
    return pl.pallas_call(
        _tc_pacc_body,
        grid=(GRID,),
        in_specs=[
            pl.BlockSpec((BN_ROWS, H2), lambda i: (i, 0)),
            pl.BlockSpec((2, BN_ROWS, H), lambda i: (0, i, 0)),
            pl.BlockSpec((H2, H2), lambda i: (0, 0)),
        ],
        out_specs=pl.BlockSpec((BN_ROWS, H2), lambda i: (i, 0)),
        out_shape=jax.ShapeDtypeStruct((N, H2), jnp.float32),
    )(p_in, h, wr1)


def _tc_final_body(p_ref, wr2_ref, br2_ref, o_ref):
    o_ref[...] = _dot(jnp.maximum(p_ref[...], 0.0), wr2_ref[...]) + br2_ref[0]


def _tc_final(p, wr2, br2):
    return pl.pallas_call(
        _tc_final_body,
        grid=(GRID,),
        in_specs=[
            pl.BlockSpec((BN_ROWS, H2), lambda i: (i, 0)),
            pl.BlockSpec((H2, OUT), lambda i: (0, 0)),
            pl.BlockSpec((1, OUT), lambda i: (0, 0)),
        ],
        out_specs=pl.BlockSpec((BN_ROWS, OUT), lambda i: (i, 0)),
        out_shape=jax.ShapeDtypeStruct((N, OUT), jnp.float32),
    )(p, wr2, br2)


def kernel(state, edge_index, c, emb, Wa, ga, ba, Wb, go, bo, Wr1, br1, Wr2,
           br2):
    src = edge_index[0]
    dst = edge_index[1]
    src_p = jnp.concatenate(
        [src, jnp.zeros((E_PAD - E,), jnp.int32)]).reshape(NTILE, NB, B)
    dst_p = jnp.concatenate(
        [dst, jnp.full((E_PAD - E,), DUMMY, jnp.int32)]).reshape(NTILE, NB, B)

    state_rs = jnp.concatenate(
        [state, jnp.zeros((NP - N,), jnp.int32)]).reshape(NTILE, NBE, BE)
    c2d = c.reshape(1, H)

    h = _sc_embed(state_rs, emb, c2d)

    wr1r = Wr1.reshape(NL, H2, H2)
    p = _tc_p0(h, wr1r[0], br1.reshape(1, H2))
    for i in range(L):
        agg = _sc_segsum(h, src_p, dst_p)
        t1, st1 = _tc_a(h, agg, Wa[i])
        t2, st2 = _tc_b(t1, st1, ga[i].reshape(1, H2), ba[i].reshape(1, H2),
                        Wb[i])
        h = _tc_c(t2, st2, go[i].reshape(1, H2), bo[i].reshape(1, H2))
        p = _tc_pacc(p, h, wr1r[i + 1])
    return _tc_final(p, Wr2, br2.reshape(1, OUT))


# speedup vs baseline: 1.2085x; 1.0239x over previous
import functools

import jax
import jax.numpy as jnp
from jax import lax
from jax.experimental import pallas as pl
from jax.experimental.pallas import tpu as pltpu
from jax.experimental.pallas import tpu_sc as plsc

N = 10000
E = 160000
H = 128
H2 = 256
L = 4
NL = 5
OUT = 128

NP = 10240
DUMMY = N
NTILE = 16
NB = 79
B = 128
EPT = NB * B
E_PAD = NTILE * EPT
RPT = NP // NTILE
BN_ROWS = 2000
GRID = N // BN_ROWS

def _sc_segsum_body(h_hbm, src_hbm, dst_hbm, out_hbm, src_v, dst_v, rows_v,
                    agg_sh):
    c = lax.axis_index("c")
    s = lax.axis_index("s")

    pltpu.sync_copy(src_hbm.at[s], src_v)
    pltpu.sync_copy(dst_hbm.at[s], dst_v)

    @pl.loop(0, B)
    def _(r):
        @pl.loop(0, H, step=16)
        def _(k):
            rows_v.at[r][pl.ds(k, 16)] = jnp.zeros((16,), jnp.float32)

    @pl.loop(0, RPT, step=B)
    def _(r0):
        pltpu.sync_copy(rows_v, agg_sh.at[pl.ds(s * RPT + r0, B)])

    plsc.subcore_barrier()

    plane = h_hbm.at[c]

    @pl.loop(0, NB)
    def _(j):
        pltpu.sync_copy(plane.at[src_v.at[j]], rows_v)
        pltpu.sync_copy(rows_v, agg_sh.at[dst_v.at[j]], add=True)

    plsc.subcore_barrier()

    pltpu.sync_copy(agg_sh.at[pl.ds(s * RPT, RPT)],
                    out_hbm.at[c].at[pl.ds(s * RPT, RPT)])


@functools.cache
def _sc_segsum_kernel():
    mesh = plsc.VectorSubcoreMesh(core_axis_name="c", subcore_axis_name="s",
                                  num_cores=2, num_subcores=NTILE)
    return pl.kernel(
        _sc_segsum_body,
        out_type=jax.ShapeDtypeStruct((2, NP, H), jnp.float32),
        mesh=mesh,
        scratch_types=[
            pltpu.VMEM((NB, B), jnp.int32),
            pltpu.VMEM((NB, B), jnp.int32),
            pltpu.VMEM((B, H), jnp.float32),
            pltpu.VMEM_SHARED((NP, H), jnp.float32),
        ],
    )


def _sc_segsum(h, src_p, dst_p):
    return _sc_segsum_kernel()(h, src_p, dst_p)


BE = 64
NBE = RPT // BE


def _sc_embed_body(state_hbm, emb_hbm, c_hbm, out_hbm, idx_v, rows_v):
    c = lax.axis_index("c")
    s = lax.axis_index("s")
    row0 = s * RPT

    @pl.when(c == 0)
    def _():
        pltpu.sync_copy(state_hbm.at[s], idx_v)

        @pl.loop(0, NBE)
        def _(b):
            pltpu.sync_copy(emb_hbm.at[idx_v.at[b]], rows_v)
            pltpu.sync_copy(rows_v,
                            out_hbm.at[0].at[pl.ds(row0 + b * BE, BE)])

    @pl.when(c == 1)
    def _():
        pltpu.sync_copy(c_hbm, rows_v.at[pl.ds(0, 1)])

        @pl.loop(1, BE)
        def _(r):
            @pl.loop(0, H, step=16)
            def _(k):
                rows_v.at[r][pl.ds(k, 16)] = rows_v.at[0][pl.ds(k, 16)]

        @pl.loop(0, NBE)
        def _(b):
            pltpu.sync_copy(rows_v,
                            out_hbm.at[1].at[pl.ds(row0 + b * BE, BE)])


@functools.cache
def _sc_embed_kernel():
    mesh = plsc.VectorSubcoreMesh(core_axis_name="c", subcore_axis_name="s",
                                  num_cores=2, num_subcores=NTILE)
    return pl.kernel(
        _sc_embed_body,
        out_type=jax.ShapeDtypeStruct((2, NP, H), jnp.float32),
        mesh=mesh,
        scratch_types=[
            pltpu.VMEM((NBE, BE), jnp.int32),
            pltpu.VMEM((BE, H), jnp.float32),
        ],
    )


def _sc_embed(state_rs, emb, c2d):
    return _sc_embed_kernel()(state_rs, emb, c2d)


_P = jax.lax.Precision.DEFAULT


def _dot(a, b):
    return jnp.dot(a, b, precision=_P, preferred_element_type=jnp.float32)


def _accum_stats(i, t, st_ref):
    @pl.when(i == 0)
    def _():
        mu0 = jnp.mean(t, axis=0)
        ctr = t - mu0[None, :]
        st_ref[...] = jnp.concatenate(
            [jnp.sum(ctr, axis=0)[None, :],
             jnp.sum(ctr * ctr, axis=0)[None, :],
             mu0[None, :]], axis=0)

    @pl.when(i > 0)
    def _():
        mu0 = st_ref[2]
        ctr = t - mu0[None, :]
        st_ref[...] += jnp.concatenate(
            [jnp.sum(ctr, axis=0)[None, :],
             jnp.sum(ctr * ctr, axis=0)[None, :],
             jnp.zeros((1, H2), jnp.float32)], axis=0)


def _tc_a_body(h_ref, agg_ref, wa_ref, t1_ref, st_ref):
    i = pl.program_id(0)
    rst = jnp.concatenate(
        [h_ref[0] + agg_ref[0], h_ref[1] + agg_ref[1]], axis=1)
    t1 = _dot(rst, wa_ref[...])
    t1_ref[...] = t1
    _accum_stats(i, t1, st_ref)


def _tc_a(h, agg, wa):
    return pl.pallas_call(
        _tc_a_body,
        grid=(GRID,),
        in_specs=[
            pl.BlockSpec((2, BN_ROWS, H), lambda i: (0, i, 0)),
            pl.BlockSpec((2, BN_ROWS, H), lambda i: (0, i, 0)),
            pl.BlockSpec((H2, H2), lambda i: (0, 0)),
        ],
        out_specs=[
            pl.BlockSpec((BN_ROWS, H2), lambda i: (i, 0)),
            pl.BlockSpec((3, H2), lambda i: (0, 0)),
        ],
        out_shape=[
            jax.ShapeDtypeStruct((N, H2), jnp.float32),
            jax.ShapeDtypeStruct((3, H2), jnp.float32),
        ],
    )(h, agg, wa)


def _bn_relu(t, st, gamma, beta):
    d1 = st[0] * (1.0 / N)
    m = st[2] + d1
    v = st[1] * (1.0 / N) - d1 * d1
    inv = lax.rsqrt(v + 1e-5)
    return jnp.maximum((t - m[None, :]) * (gamma * inv)[None, :] + beta[None, :],
                       0.0)


def _tc_b_body(t1_ref, st_ref, ga_ref, ba_ref, wb_ref, t2_ref, st2_ref):
    i = pl.program_id(0)
    z = _bn_relu(t1_ref[...], st_ref[...], ga_ref[0], ba_ref[0])
    t2 = _dot(z, wb_ref[...])
    t2_ref[...] = t2
    _accum_stats(i, t2, st2_ref)


def _tc_b(t1, st1, ga, ba, wb):
    return pl.pallas_call(
        _tc_b_body,
        grid=(GRID,),
        in_specs=[
            pl.BlockSpec((BN_ROWS, H2), lambda i: (i, 0)),
            pl.BlockSpec((3, H2), lambda i: (0, 0)),
            pl.BlockSpec((1, H2), lambda i: (0, 0)),
            pl.BlockSpec((1, H2), lambda i: (0, 0)),
            pl.BlockSpec((H2, H2), lambda i: (0, 0)),
        ],
        out_specs=[
            pl.BlockSpec((BN_ROWS, H2), lambda i: (i, 0)),
            pl.BlockSpec((3, H2), lambda i: (0, 0)),
        ],
        out_shape=[
            jax.ShapeDtypeStruct((N, H2), jnp.float32),
            jax.ShapeDtypeStruct((3, H2), jnp.float32),
        ],
    )(t1, st1, ga, ba, wb)


def _tc_c_body(t2_ref, st_ref, go_ref, bo_ref, h_ref):
    h = _bn_relu(t2_ref[...], st_ref[...], go_ref[0], bo_ref[0])
    h_ref[0] = h[:, :H]
    h_ref[1] = h[:, H:]


def _tc_c(t2, st2, go, bo):
    return pl.pallas_call(
        _tc_c_body,
        grid=(GRID,),
        in_specs=[
            pl.BlockSpec((BN_ROWS, H2), lambda i: (i, 0)),
            pl.BlockSpec((3, H2), lambda i: (0, 0)),
            pl.BlockSpec((1, H2), lambda i: (0, 0)),
            pl.BlockSpec((1, H2), lambda i: (0, 0)),
        ],
        out_specs=pl.BlockSpec((2, BN_ROWS, H), lambda i: (0, i, 0)),
        out_shape=jax.ShapeDtypeStruct((2, NP, H), jnp.float32),
    )(t2, st2, go, bo)


def _tc_readout_body(h0, h1, h2, h3, h4, wr1_ref, br1_ref, wr2_ref, br2_ref,
                     o_ref):
    acc = jnp.broadcast_to(br1_ref[0], (BN_ROWS, H2))
    for k, h_ref in enumerate((h0, h1, h2, h3, h4)):
        hcat = jnp.concatenate([h_ref[0], h_ref[1]], axis=1)
        acc = acc + _dot(hcat, wr1_ref[k])
    o_ref[...] = _dot(jnp.maximum(acc, 0.0), wr2_ref[...]) + br2_ref[0]


def _tc_readout(hs, wr1r, br1, wr2, br2):
    hspec = pl.BlockSpec((2, BN_ROWS, H), lambda i: (0, i, 0))
    return pl.pallas_call(
        _tc_readout_body,
        grid=(GRID,),
        in_specs=[hspec] * NL + [
            pl.BlockSpec((NL, H2, H2), lambda i: (0, 0, 0)),
            pl.BlockSpec((1, H2), lambda i: (0, 0)),
            pl.BlockSpec((H2, OUT), lambda i: (0, 0)),
            pl.BlockSpec((1, OUT), lambda i: (0, 0)),
        ],
        out_specs=pl.BlockSpec((BN_ROWS, OUT), lambda i: (i, 0)),
        out_shape=jax.ShapeDtypeStruct((N, OUT), jnp.float32),
    )(*hs, wr1r, br1, wr2, br2)


def kernel(state, edge_index, c, emb, Wa, ga, ba, Wb, go, bo, Wr1, br1, Wr2,
           br2):
    src = edge_index[0]
    dst = edge_index[1]
    src_p = jnp.concatenate(
        [src, jnp.zeros((E_PAD - E,), jnp.int32)]).reshape(NTILE, NB, B)
    dst_p = jnp.concatenate(
        [dst, jnp.full((E_PAD - E,), DUMMY, jnp.int32)]).reshape(NTILE, NB, B)

    state_rs = jnp.concatenate(
        [state, jnp.zeros((NP - N,), jnp.int32)]).reshape(NTILE, NBE, BE)
    c2d = c.reshape(1, H)

    h = _sc_embed(state_rs, emb, c2d)

    hs = [h]
    for i in range(L):
        agg = _sc_segsum(h, src_p, dst_p)
        t1, st1 = _tc_a(h, agg, Wa[i])
        t2, st2 = _tc_b(t1, st1, ga[i].reshape(1, H2), ba[i].reshape(1, H2),
                        Wb[i])
        h = _tc_c(t2, st2, go[i].reshape(1, H2), bo[i].reshape(1, H2))
        hs.append(h)

    wr1r = Wr1.reshape(NL, H2, H2)
    return _tc_readout(hs, wr1r, br1.reshape(1, H2), Wr2, br2.reshape(1, OUT))
